# baseline (speedup 1.0000x reference)
import functools

import jax
import jax.numpy as jnp
from jax import lax
from jax.experimental import pallas as pl
from jax.experimental.pallas import tpu as pltpu
from jax.experimental.pallas import tpu_sc as plsc

_NBUF = 4
_CB = 128
_TP = 133
_J = 4


@functools.cache
def _make_lookup(S0, S1, V, D):
    info = plsc.get_sparse_core_info()
    NC, NS = info.num_cores, info.num_subcores
    NW = NC * NS
    L = info.num_lanes
    assert S0 % (NW * _CB) == 0 and D % 8 == 0 and _CB % L == 0
    C = S0 // NW
    H = C // _CB
    n = S1 * H
    DT = D // 8
    BT = S0 // 128
    assert n % _J == 0 and n >= 3 * _J
    mesh = plsc.VectorSubcoreMesh(core_axis_name="c", subcore_axis_name="s")

    @functools.partial(
        pl.kernel,
        mesh=mesh,
        out_type=jax.ShapeDtypeStruct((S1, DT, BT, 8, 128), jnp.float32),
        scratch_types=[
            pltpu.VMEM((S1, C), jnp.int32),
            *[pltpu.VMEM((_CB, D), jnp.float32) for _ in range(_NBUF)],
            *[pltpu.VMEM((D, _TP), jnp.float32) for _ in range(2)],
            *[pltpu.SemaphoreType.DMA for _ in range(_NBUF + 2)],
        ],
        compiler_params=pltpu.CompilerParams(
            use_tc_tiling_on_sc=False, needs_layout_passes=False
        ),
    )
    def k(idx_hbm, table_hbm, out_hbm, idx_v, *rest):
        rows = rest[:_NBUF]
        tb = rest[_NBUF : _NBUF + 2]
        gs = rest[_NBUF + 2 : 2 * _NBUF + 2]
        ws = rest[2 * _NBUF + 2 :]
        wid = lax.axis_index("s") * NC + lax.axis_index("c")
        b0 = wid * C
        bt0 = wid * H
        pltpu.sync_copy(idx_hbm.at[:, pl.ds(b0, C)], idx_v)

        lane = lax.iota(jnp.int32, L)
        halves = [lane + h * L for h in range(D // L)]

        def gather(t, slot):
            s = t // H
            h = t - s * H
            return pltpu.async_copy(
                table_hbm.at[idx_v.at[s, pl.ds(h * _CB, _CB)]],
                rows[slot],
                gs[slot],
            )

        def wait_gather(slot):
            pltpu.make_async_copy(
                table_hbm.at[idx_v.at[0, pl.ds(0, _CB)]], rows[slot], gs[slot]
            ).wait()

        def put(t, slot):
            s = t // H
            h = t - s * H
            cps = []
            for dt in range(DT):
                cps.append(
                    pltpu.async_copy(
                        tb[slot].at[pl.ds(dt * 8, 8), pl.ds(0, 128)],
                        out_hbm.at[s, dt, bt0 + h],
                        ws[slot],
                    )
                )
            return cps

        def wait_put(slot):
            for dt in range(DT):
                pltpu.make_async_copy(
                    tb[slot].at[pl.ds(dt * 8, 8), pl.ds(0, 128)],
                    out_hbm.at[0, 0, 0],
                    ws[slot],
                ).wait()

        def transpose(rslot, tslot):
            src = rows[rslot]
            dst = tb[tslot]
            for h in range(D // L):
                d_all = halves[h]
                for b in range(_CB):
                    v = src[b, pl.ds(h * L, L)]
                    plsc.store_scatter(dst, [d_all, jnp.full((L,), b, jnp.int32)], v)

        for t in range(_NBUF):
            gather(t, t)
        for t in range(_J):
            wait_gather(t % _NBUF)
            if t >= 2:
                wait_put(t % 2)
            transpose(t % _NBUF, t % 2)
            put(t, t % 2)
            gather(t + _NBUF, t % _NBUF)

        def body(i, carry):
            t0 = i * _J
            for j in range(_J):
                t = t0 + j
                slot = j % _NBUF
                ts = j % 2
                wait_gather(slot)
                wait_put(ts)
                transpose(slot, ts)
                put(t, ts)
                nt = jnp.minimum(t + _NBUF, n - 1)
                gather(nt, slot)
            return carry

        lax.fori_loop(1, n // _J - 1, body, 0)

        for t in range(n - _J, n):
            wait_gather(t % _NBUF)
            wait_put(t % 2)
            transpose(t % _NBUF, t % 2)
            put(t, t % 2)
        wait_put(0)
        wait_put(1)

    return k


def kernel(indices, weight):
    S0, S1 = indices.shape
    V, D = weight.shape
    r = _make_lookup(S0, S1, V, D)(indices.T, weight)
    return r.transpose(2, 4, 0, 1, 3).reshape(S0, S1, D)


# speedup vs baseline: 1.1279x; 1.1279x over previous
import functools

import jax
import jax.numpy as jnp
from jax import lax
from jax.experimental import pallas as pl
from jax.experimental.pallas import tpu as pltpu
from jax.experimental.pallas import tpu_sc as plsc

_NBUF = 10
_CB = 128
_TP = 133
_J = 10


@functools.cache
def _make_lookup(S0, S1, V, D):
    info = plsc.get_sparse_core_info()
    NC, NS = info.num_cores, info.num_subcores
    NW = NC * NS
    L = info.num_lanes
    assert S0 % (NW * _CB) == 0 and D % 8 == 0 and _CB % L == 0
    C = S0 // NW
    H = C // _CB
    n = S1 * H
    DT = D // 8
    BT = S0 // 128
    assert n % _J == 0 and n >= 3 * _J
    mesh = plsc.VectorSubcoreMesh(core_axis_name="c", subcore_axis_name="s")

    @functools.partial(
        pl.kernel,
        mesh=mesh,
        out_type=jax.ShapeDtypeStruct((S1, DT, BT, 8, 128), jnp.float32),
        scratch_types=[
            pltpu.VMEM((S1, C), jnp.int32),
            *[pltpu.VMEM((_CB, D), jnp.float32) for _ in range(_NBUF)],
            *[pltpu.VMEM((DT, 8, _TP), jnp.float32) for _ in range(2)],
            *[pltpu.SemaphoreType.DMA for _ in range(_NBUF + 2)],
        ],
        compiler_params=pltpu.CompilerParams(
            use_tc_tiling_on_sc=False, needs_layout_passes=False
        ),
    )
    def k(idx_hbm, table_hbm, out_hbm, idx_v, *rest):
        rows = rest[:_NBUF]
        tb = rest[_NBUF : _NBUF + 2]
        gs = rest[_NBUF + 2 : 2 * _NBUF + 2]
        ws = rest[2 * _NBUF + 2 :]
        wid = lax.axis_index("s") * NC + lax.axis_index("c")
        b0 = wid * C
        bt0 = wid * H
        pltpu.sync_copy(idx_hbm.at[:, pl.ds(b0, C)], idx_v)

        lane = lax.iota(jnp.int32, L)
        halves = [((lane + h * L) >> 3, (lane + h * L) & 7) for h in range(D // L)]

        def gather(t, slot):
            s = t // H
            h = t - s * H
            return pltpu.async_copy(
                table_hbm.at[idx_v.at[s, pl.ds(h * _CB, _CB)]],
                rows[slot],
                gs[slot],
            )

        def wait_gather(slot):
            pltpu.make_async_copy(
                table_hbm.at[idx_v.at[0, pl.ds(0, _CB)]], rows[slot], gs[slot]
            ).wait()

        def put(t, slot):
            s = t // H
            h = t - s * H
            return pltpu.async_copy(
                tb[slot].at[:, :, pl.ds(0, 128)],
                out_hbm.at[s, :, bt0 + h],
                ws[slot],
            )

        def wait_put(slot):
            pltpu.make_async_copy(
                tb[slot].at[:, :, pl.ds(0, 128)],
                out_hbm.at[0, :, 0],
                ws[slot],
            ).wait()

        def transpose(rslot, tslot):
            src = rows[rslot]
            dst = tb[tslot]
            zero = jnp.zeros((L,), jnp.int32)

            def tb_body(bg, carry):
                for bb in range(8):
                    b = bg * 8 + bb
                    bi = zero + b
                    for h in range(D // L):
                        v = src[b, pl.ds(h * L, L)]
                        i0, i1 = halves[h]
                        plsc.store_scatter(dst, [i0, i1, bi], v)
                return carry

            lax.fori_loop(0, _CB // 8, tb_body, 0)

        for t in range(_NBUF):
            gather(t, t)
        for t in range(_J):
            wait_gather(t % _NBUF)
            if t >= 2:
                wait_put(t % 2)
            transpose(t % _NBUF, t % 2)
            put(t, t % 2)
            gather(t + _NBUF, t % _NBUF)

        def body(i, carry):
            t0 = i * _J
            for j in range(_J):
                t = t0 + j
                slot = j % _NBUF
                ts = j % 2
                wait_gather(slot)
                wait_put(ts)
                transpose(slot, ts)
                put(t, ts)
                nt = jnp.minimum(t + _NBUF, n - 1)
                gather(nt, slot)
            return carry

        lax.fori_loop(1, n // _J - 1, body, 0)

        for t in range(n - _J, n):
            wait_gather(t % _NBUF)
            wait_put(t % 2)
            transpose(t % _NBUF, t % 2)
            put(t, t % 2)
        wait_put(0)
        wait_put(1)

    return k


def kernel(indices, weight):
    S0, S1 = indices.shape
    V, D = weight.shape
    r = _make_lookup(S0, S1, V, D)(indices.T, weight)
    return r.transpose(2, 4, 0, 1, 3).reshape(S0, S1, D)
